# baseline (speedup 1.0000x reference)
import functools

import jax
import jax.numpy as jnp
from jax import lax
from jax.experimental import pallas as pl
from jax.experimental.pallas import tpu as pltpu
import jax.experimental.pallas.tpu_sc as plsc

N = 320000
D = 128
S = 2048
NC = 2
NS = 16
NW = NC * NS
C = 128
NBUF = 4
PD = 2
ACC_ROWS = S + 8
ZROWS = ACC_ROWS // NS // 8 * 8
SEG_PT = S // NW
LANES = 16


def _tile_body(nodes_hbm, batch_hbm, bounds_hbm, out_hbm,
               rows_v, ids_v, lidx_v, bnd_v, zbuf_v, acc_sh, *sems):
    rsems = sems[:NBUF]
    isems = sems[NBUF:2 * NBUF]
    ssems = sems[2 * NBUF:]
    sid = lax.axis_index("s")
    cid = lax.axis_index("c")
    wid = cid * NS + sid
    lanes = lax.iota(jnp.int32, 16)

    pltpu.sync_copy(bounds_hbm.at[pl.ds(wid, 1)], bnd_v)
    bvec = bnd_v[0, :]
    start = bvec[0]
    end = bvec[1]

    astart = jnp.bitwise_and(start, jnp.int32(-8))
    nchunks = lax.shift_right_arithmetic(end - astart + (C - 1), 7)
    nouter = lax.shift_right_arithmetic(nchunks + (NBUF - 1), 2)

    def chunk_base(k):
        return pl.multiple_of(jnp.minimum(astart + k * C, N - C), 8)

    def fetch(k, b):
        base = chunk_base(k)
        pltpu.async_copy(batch_hbm.at[pl.ds(base, C)], ids_v.at[b], isems[b])
        pltpu.async_copy(nodes_hbm.at[pl.ds(base, C)], rows_v.at[b], rsems[b])

    def wait_fetch(k, b):
        base = chunk_base(k)
        pltpu.make_async_copy(
            batch_hbm.at[pl.ds(base, C)], ids_v.at[b], isems[b]).wait()
        pltpu.make_async_copy(
            nodes_hbm.at[pl.ds(base, C)], rows_v.at[b], rsems[b]).wait()

    def scatter(b):
        pltpu.async_copy(rows_v.at[b], acc_sh.at[lidx_v.at[b]],
                         ssems[b], add=True)

    def wait_scatter(b):
        pltpu.make_async_copy(
            rows_v.at[b], acc_sh.at[lidx_v.at[b]], ssems[b]).wait()

    for b in range(PD):
        fetch(jnp.int32(b), b)

    zz = jnp.zeros((LANES,), jnp.float32)

    def zero_row(i, carry):
        for j in range(D // LANES):
            zbuf_v[i, pl.ds(j * LANES, LANES)] = zz
        return carry

    lax.fori_loop(0, ZROWS, zero_row, 0)
    pltpu.sync_copy(zbuf_v, acc_sh.at[pl.ds(sid * ZROWS, ZROWS)])

    @pl.when(sid == 0)
    def _():
        pltpu.sync_copy(zbuf_v.at[pl.ds(0, ACC_ROWS - S)],
                        acc_sh.at[pl.ds(S, ACC_ROWS - S)])

    plsc.subcore_barrier()

    def outer(k0, carry):
        for b in range(NBUF):
            k = k0 * NBUF + b
            nominal = astart + k * C
            lo = jnp.maximum(start, nominal)
            hi = jnp.minimum(end, nominal + C)
            base = chunk_base(k)
            wait_fetch(k, b)
            b2 = (b + PD) % NBUF
            if b in (0, 1):
                @pl.when(k0 > 0)
                def _():
                    wait_scatter(b2)
            else:
                wait_scatter(b2)
            for g in range(C // LANES):
                rg = base + (g * LANES) + lanes
                idv = ids_v[b, pl.ds(g * LANES, LANES)]
                keep = (rg >= lo) & (rg < hi)
                lidx_v[b, pl.ds(g * LANES, LANES)] = jnp.where(
                    keep, idv, jnp.int32(S))
            scatter(b)
            fetch(k + PD, b2)
        return carry

    lax.fori_loop(0, nouter, outer, 0)

    T = nouter * NBUF
    for b in range(PD):
        wait_fetch(T + b, b)

    @pl.when(nouter > 0)
    def _():
        wait_scatter(NBUF - 2)
        wait_scatter(NBUF - 1)

    plsc.subcore_barrier()
    off = cid * (S // NC) + sid * SEG_PT
    pltpu.sync_copy(acc_sh.at[pl.ds(off, SEG_PT)],
                    out_hbm.at[pl.ds(off, SEG_PT)])


@functools.partial(
    pl.kernel,
    out_type=jax.ShapeDtypeStruct((S, D), jnp.float32),
    mesh=plsc.VectorSubcoreMesh(core_axis_name="c", subcore_axis_name="s"),
    scratch_types=[
        pltpu.VMEM((NBUF, C, D), jnp.float32),
        pltpu.VMEM((NBUF, C), jnp.int32),
        pltpu.VMEM((NBUF, C), jnp.int32),
        pltpu.VMEM((1, 16), jnp.int32),
        pltpu.VMEM((ZROWS, D), jnp.float32),
        pltpu.MemorySpace.VMEM_SHARED((ACC_ROWS, D), jnp.float32),
    ] + [pltpu.SemaphoreType.DMA] * (3 * NBUF),
)
def _segment_sum_sc(nodes_hbm, batch_hbm, bounds_hbm, out_hbm,
                    rows_v, ids_v, lidx_v, bnd_v, zbuf_v, acc_sh, *sems):
    _tile_body(nodes_hbm, batch_hbm, bounds_hbm, out_hbm,
               rows_v, ids_v, lidx_v, bnd_v, zbuf_v, acc_sh, *sems)


def kernel(node_features, batch, ptr):
    rsplit = jnp.searchsorted(
        batch, jnp.int32(S // NC), side="left").astype(jnp.int32)
    i = jnp.arange(NS + 1, dtype=jnp.int32)
    s0 = rsplit * i // NS
    s1 = rsplit + (N - rsplit) * i // NS
    starts = jnp.concatenate([s0[:-1], s1[:-1], jnp.array([N], jnp.int32)])
    bounds = jnp.zeros((NW, 16), jnp.int32)
    bounds = bounds.at[:, 0].set(starts[:-1]).at[:, 1].set(starts[1:])
    return _segment_sum_sc(node_features, batch, bounds)


# speedup vs baseline: 1.7307x; 1.7307x over previous
import functools

import jax
import jax.numpy as jnp
from jax import lax
from jax.experimental import pallas as pl
from jax.experimental.pallas import tpu as pltpu
import jax.experimental.pallas.tpu_sc as plsc

N = 320000
D = 128
S = 2048
NC = 2
NS = 16
NW = NC * NS
C = 80
NCHUNKS = 90
RPT = NCHUNKS * C
NSC = NW * RPT
NBUF = 5
PD = 3
NOUTER = NCHUNKS // NBUF
SROWS = S // NS
LANES = 16

R = 2560
B = (N - NSC) // R
W = 32


def _tile_body(nodes_hbm, batch2d_hbm, pout_hbm,
               rows_v, ids_v, zbuf_v, acc_sh, *sems):
    rsems = sems[:NBUF]
    ssems = sems[NBUF:]
    sid = lax.axis_index("s")
    cid = lax.axis_index("c")
    wid = cid * NS + sid
    row0 = wid * RPT

    def chunk_base(k):
        return pl.multiple_of(jnp.minimum(row0 + k * C, N - C), 8)

    def fetch(k, b):
        pltpu.async_copy(nodes_hbm.at[pl.ds(chunk_base(k), C)],
                         rows_v.at[b], rsems[b])

    def wait_fetch(k, b):
        pltpu.make_async_copy(
            nodes_hbm.at[pl.ds(chunk_base(k), C)],
            rows_v.at[b], rsems[b]).wait()

    def scatter(k, b):
        pltpu.async_copy(rows_v.at[b], acc_sh.at[ids_v.at[k]],
                         ssems[b], add=True)

    def wait_scatter(k, b):
        pltpu.make_async_copy(
            rows_v.at[b], acc_sh.at[ids_v.at[k]], ssems[b]).wait()

    pltpu.sync_copy(batch2d_hbm.at[wid], ids_v)
    for b in range(PD):
        fetch(jnp.int32(b), b)

    zz = jnp.zeros((LANES,), jnp.float32)

    def zero_row(i, carry):
        for j in range(D // LANES):
            zbuf_v[i, pl.ds(j * LANES, LANES)] = zz
        return carry

    lax.fori_loop(0, SROWS, zero_row, 0)
    pltpu.sync_copy(zbuf_v, acc_sh.at[pl.ds(sid * SROWS, SROWS)])
    plsc.subcore_barrier()

    def outer(k0, carry):
        for b in range(NBUF):
            k = k0 * NBUF + b
            wait_fetch(k, b)
            b2 = (b + PD) % NBUF
            if b in (0, 1):
                @pl.when(k0 > 0)
                def _():
                    wait_scatter(k - 2, b2)
            else:
                wait_scatter(k - 2, b2)
            scatter(k, b)
            fetch(k + PD, b2)
        return carry

    lax.fori_loop(0, NOUTER, outer, 0)

    for b in range(PD):
        wait_fetch(NCHUNKS + b, b)
    wait_scatter(NCHUNKS - 2, NBUF - 2)
    wait_scatter(NCHUNKS - 1, NBUF - 1)

    plsc.subcore_barrier()
    pltpu.sync_copy(acc_sh.at[pl.ds(sid * SROWS, SROWS)],
                    pout_hbm.at[cid].at[pl.ds(sid * SROWS, SROWS)])


@functools.partial(
    pl.kernel,
    out_type=jax.ShapeDtypeStruct((NC, S, D), jnp.float32),
    mesh=plsc.VectorSubcoreMesh(core_axis_name="c", subcore_axis_name="s"),
    scratch_types=[
        pltpu.VMEM((NBUF, C, D), jnp.float32),
        pltpu.VMEM((NCHUNKS, C), jnp.int32),
        pltpu.VMEM((SROWS, D), jnp.float32),
        pltpu.MemorySpace.VMEM_SHARED((S, D), jnp.float32),
    ] + [pltpu.SemaphoreType.DMA] * (2 * NBUF),
)
def _segment_sum_sc(nodes_hbm, batch2d_hbm, pout_hbm,
                    rows_v, ids_v, zbuf_v, acc_sh, *sems):
    _tile_body(nodes_hbm, batch2d_hbm, pout_hbm,
               rows_v, ids_v, zbuf_v, acc_sh, *sems)


def _tc_body(ids_ref, x_ref, o_ref):
    @pl.when(pl.program_id(0) == 0)
    def _():
        o_ref[...] = jnp.zeros((S, D), jnp.float32)

    idv = ids_ref[0]
    x = x_ref[...]
    first = jnp.min(idv)
    last = jnp.max(idv)
    wiota = lax.broadcasted_iota(jnp.int32, (W, R), 0)

    def cond(ws):
        return ws <= last

    def body(ws):
        lws = jnp.minimum(ws, S - W)
        oh = (idv + jnp.zeros((W, R), jnp.int32) == lws + wiota)
        win = jax.lax.dot_general(
            oh.astype(jnp.float32), x, (((1,), (0,)), ((), ())),
            preferred_element_type=jnp.float32)
        o_ref[pl.ds(lws, W), :] += win
        nxt = jnp.min(jnp.where(idv >= lws + W, idv, jnp.int32(S + W)))
        return nxt

    lax.while_loop(cond, body, first)


def _tc_partial(ids_tc, rows_tc):
    return pl.pallas_call(
        _tc_body,
        grid=(B,),
        in_specs=[
            pl.BlockSpec((1, 1, R), lambda i: (i, 0, 0)),
            pl.BlockSpec((R, D), lambda i: (NSC // R + i, 0)),
        ],
        out_specs=pl.BlockSpec((S, D), lambda i: (0, 0)),
        out_shape=jax.ShapeDtypeStruct((S, D), jnp.float32),
    )(ids_tc, rows_tc)


def _combine_body(p_ref, t_ref, o_ref):
    o_ref[...] = p_ref[0] + p_ref[1] + t_ref[...]


def _combine(partials, ptc):
    blk = 256
    return pl.pallas_call(
        _combine_body,
        grid=(S // blk,),
        in_specs=[
            pl.BlockSpec((NC, blk, D), lambda i: (0, i, 0)),
            pl.BlockSpec((blk, D), lambda i: (i, 0)),
        ],
        out_specs=pl.BlockSpec((blk, D), lambda i: (i, 0)),
        out_shape=jax.ShapeDtypeStruct((S, D), jnp.float32),
    )(partials, ptc)


def kernel(node_features, batch, ptr):
    partials = _segment_sum_sc(
        node_features, batch[:NSC].reshape(NW, NCHUNKS, C))
    ptc = _tc_partial(batch[NSC:].reshape(B, 1, R), node_features)
    return _combine(partials, ptc)


# speedup vs baseline: 1.7487x; 1.0104x over previous
import functools

import jax
import jax.numpy as jnp
from jax import lax
from jax.experimental import pallas as pl
from jax.experimental.pallas import tpu as pltpu
import jax.experimental.pallas.tpu_sc as plsc

N = 320000
D = 128
S = 2048
NC = 2
NS = 16
NW = NC * NS
C = 80
NCHUNKS = 90
RPT = NCHUNKS * C
NSC = NW * RPT
NBUF = 5
PD = 3
NOUTER = NCHUNKS // NBUF
SROWS = S // NS
LANES = 16

R = 2560
B = (N - NSC) // R
W = 32


def _tile_body(nodes_hbm, batch2d_hbm, pout_hbm,
               rows_v, ids_v, zbuf_v, acc_sh, *sems):
    rsems = sems[:NBUF]
    ssems = sems[NBUF:]
    sid = lax.axis_index("s")
    cid = lax.axis_index("c")
    wid = cid * NS + sid
    row0 = wid * RPT

    def chunk_base(k):
        return pl.multiple_of(jnp.minimum(row0 + k * C, N - C), 8)

    def fetch(k, b):
        pltpu.async_copy(nodes_hbm.at[pl.ds(chunk_base(k), C)],
                         rows_v.at[b], rsems[b])

    def wait_fetch(k, b):
        pltpu.make_async_copy(
            nodes_hbm.at[pl.ds(chunk_base(k), C)],
            rows_v.at[b], rsems[b]).wait()

    def scatter(k, b):
        pltpu.async_copy(rows_v.at[b], acc_sh.at[ids_v.at[k]],
                         ssems[b], add=True)

    def wait_scatter(k, b):
        pltpu.make_async_copy(
            rows_v.at[b], acc_sh.at[ids_v.at[k]], ssems[b]).wait()

    pltpu.sync_copy(batch2d_hbm.at[wid], ids_v)
    for b in range(PD):
        fetch(jnp.int32(b), b)

    zz = jnp.zeros((LANES,), jnp.float32)

    def zero_row(i, carry):
        for j in range(D // LANES):
            zbuf_v[i, pl.ds(j * LANES, LANES)] = zz
        return carry

    lax.fori_loop(0, SROWS, zero_row, 0)
    pltpu.sync_copy(zbuf_v, acc_sh.at[pl.ds(sid * SROWS, SROWS)])
    plsc.subcore_barrier()

    def outer(k0, carry):
        for b in range(NBUF):
            k = k0 * NBUF + b
            wait_fetch(k, b)
            b2 = (b + PD) % NBUF
            if b in (0, 1):
                @pl.when(k0 > 0)
                def _():
                    wait_scatter(k - 2, b2)
            else:
                wait_scatter(k - 2, b2)
            scatter(k, b)
            fetch(k + PD, b2)
        return carry

    lax.fori_loop(0, NOUTER, outer, 0)

    for b in range(PD):
        wait_fetch(NCHUNKS + b, b)
    wait_scatter(NCHUNKS - 2, NBUF - 2)
    wait_scatter(NCHUNKS - 1, NBUF - 1)

    plsc.subcore_barrier()
    pltpu.sync_copy(acc_sh.at[pl.ds(sid * SROWS, SROWS)],
                    pout_hbm.at[cid].at[pl.ds(sid * SROWS, SROWS)])


@functools.partial(
    pl.kernel,
    out_type=jax.ShapeDtypeStruct((NC, S, D), jnp.float32),
    mesh=plsc.VectorSubcoreMesh(core_axis_name="c", subcore_axis_name="s"),
    scratch_types=[
        pltpu.VMEM((NBUF, C, D), jnp.float32),
        pltpu.VMEM((NCHUNKS, C), jnp.int32),
        pltpu.VMEM((SROWS, D), jnp.float32),
        pltpu.MemorySpace.VMEM_SHARED((S, D), jnp.float32),
    ] + [pltpu.SemaphoreType.DMA] * (2 * NBUF),
)
def _segment_sum_sc(nodes_hbm, batch2d_hbm, pout_hbm,
                    rows_v, ids_v, zbuf_v, acc_sh, *sems):
    _tile_body(nodes_hbm, batch2d_hbm, pout_hbm,
               rows_v, ids_v, zbuf_v, acc_sh, *sems)


def _tc_body(ids_ref, x_ref, o_ref):
    @pl.when(pl.program_id(0) == 0)
    def _():
        o_ref[...] = jnp.zeros((S, D), jnp.float32)

    idv = ids_ref[0]
    x = x_ref[...]
    first = jnp.min(idv)
    last = jnp.max(idv)
    wiota = lax.broadcasted_iota(jnp.int32, (W, R), 0)

    def cond(ws):
        return ws <= last

    def body(ws):
        lws = jnp.minimum(ws, S - W)
        oh = (idv + jnp.zeros((W, R), jnp.int32) == lws + wiota)
        win = jax.lax.dot_general(
            oh.astype(jnp.float32), x, (((1,), (0,)), ((), ())),
            precision=jax.lax.Precision.HIGHEST,
            preferred_element_type=jnp.float32)
        o_ref[pl.ds(lws, W), :] += win
        nxt = jnp.min(jnp.where(idv >= lws + W, idv, jnp.int32(S + W)))
        return nxt

    lax.while_loop(cond, body, first)


def _tc_partial(ids_tc, rows_tc):
    return pl.pallas_call(
        _tc_body,
        grid=(B,),
        in_specs=[
            pl.BlockSpec((1, 1, R), lambda i: (i, 0, 0)),
            pl.BlockSpec((R, D), lambda i: (NSC // R + i, 0)),
        ],
        out_specs=pl.BlockSpec((S, D), lambda i: (0, 0)),
        out_shape=jax.ShapeDtypeStruct((S, D), jnp.float32),
    )(ids_tc, rows_tc)


def _combine_body(p_ref, t_ref, o_ref):
    o_ref[...] = p_ref[0] + p_ref[1] + t_ref[...]


def _combine(partials, ptc):
    blk = 256
    return pl.pallas_call(
        _combine_body,
        grid=(S // blk,),
        in_specs=[
            pl.BlockSpec((NC, blk, D), lambda i: (0, i, 0)),
            pl.BlockSpec((blk, D), lambda i: (i, 0)),
        ],
        out_specs=pl.BlockSpec((blk, D), lambda i: (i, 0)),
        out_shape=jax.ShapeDtypeStruct((S, D), jnp.float32),
    )(partials, ptc)


def kernel(node_features, batch, ptr):
    partials = _segment_sum_sc(
        node_features, batch[:NSC].reshape(NW, NCHUNKS, C))
    ptc = _tc_partial(batch[NSC:].reshape(B, 1, R), node_features)
    return _combine(partials, ptc)


# speedup vs baseline: 1.8569x; 1.0618x over previous
import functools

import jax
import jax.numpy as jnp
from jax import lax
from jax.experimental import pallas as pl
from jax.experimental.pallas import tpu as pltpu
import jax.experimental.pallas.tpu_sc as plsc

N = 320000
D = 128
S = 2048
NC = 2
NS = 16
NW = NC * NS
C = 80
NCHUNKS = 75
RPT = NCHUNKS * C
NSC = NW * RPT
NBUF = 5
PD = 3
NOUTER = NCHUNKS // NBUF
SROWS = S // NS
LANES = 16

R = 6400
B = (N - NSC) // R
W = 48


def _tile_body(nodes_hbm, batch2d_hbm, pout_hbm,
               rows_v, ids_v, zbuf_v, acc_sh, *sems):
    rsems = sems[:NBUF]
    ssems = sems[NBUF:]
    sid = lax.axis_index("s")
    cid = lax.axis_index("c")
    wid = cid * NS + sid
    row0 = wid * RPT

    def chunk_base(k):
        return pl.multiple_of(jnp.minimum(row0 + k * C, N - C), 8)

    def fetch(k, b):
        pltpu.async_copy(nodes_hbm.at[pl.ds(chunk_base(k), C)],
                         rows_v.at[b], rsems[b])

    def wait_fetch(k, b):
        pltpu.make_async_copy(
            nodes_hbm.at[pl.ds(chunk_base(k), C)],
            rows_v.at[b], rsems[b]).wait()

    def scatter(k, b):
        pltpu.async_copy(rows_v.at[b], acc_sh.at[ids_v.at[k]],
                         ssems[b], add=True)

    def wait_scatter(k, b):
        pltpu.make_async_copy(
            rows_v.at[b], acc_sh.at[ids_v.at[k]], ssems[b]).wait()

    pltpu.sync_copy(batch2d_hbm.at[wid], ids_v)
    for b in range(PD):
        fetch(jnp.int32(b), b)

    zz = jnp.zeros((LANES,), jnp.float32)

    def zero_row(i, carry):
        for j in range(D // LANES):
            zbuf_v[i, pl.ds(j * LANES, LANES)] = zz
        return carry

    lax.fori_loop(0, SROWS, zero_row, 0)
    pltpu.sync_copy(zbuf_v, acc_sh.at[pl.ds(sid * SROWS, SROWS)])
    plsc.subcore_barrier()

    def outer(k0, carry):
        for b in range(NBUF):
            k = k0 * NBUF + b
            wait_fetch(k, b)
            b2 = (b + PD) % NBUF
            if b in (0, 1):
                @pl.when(k0 > 0)
                def _():
                    wait_scatter(k - 2, b2)
            else:
                wait_scatter(k - 2, b2)
            scatter(k, b)
            fetch(k + PD, b2)
        return carry

    lax.fori_loop(0, NOUTER, outer, 0)

    for b in range(PD):
        wait_fetch(NCHUNKS + b, b)
    wait_scatter(NCHUNKS - 2, NBUF - 2)
    wait_scatter(NCHUNKS - 1, NBUF - 1)

    plsc.subcore_barrier()
    pltpu.sync_copy(acc_sh.at[pl.ds(sid * SROWS, SROWS)],
                    pout_hbm.at[cid].at[pl.ds(sid * SROWS, SROWS)])


@functools.partial(
    pl.kernel,
    out_type=jax.ShapeDtypeStruct((NC, S, D), jnp.float32),
    mesh=plsc.VectorSubcoreMesh(core_axis_name="c", subcore_axis_name="s"),
    scratch_types=[
        pltpu.VMEM((NBUF, C, D), jnp.float32),
        pltpu.VMEM((NCHUNKS, C), jnp.int32),
        pltpu.VMEM((SROWS, D), jnp.float32),
        pltpu.MemorySpace.VMEM_SHARED((S, D), jnp.float32),
    ] + [pltpu.SemaphoreType.DMA] * (2 * NBUF),
)
def _segment_sum_sc(nodes_hbm, batch2d_hbm, pout_hbm,
                    rows_v, ids_v, zbuf_v, acc_sh, *sems):
    _tile_body(nodes_hbm, batch2d_hbm, pout_hbm,
               rows_v, ids_v, zbuf_v, acc_sh, *sems)


def _tc_body(ids_ref, x_ref, o_ref):
    @pl.when(pl.program_id(0) == 0)
    def _():
        o_ref[...] = jnp.zeros((S, D), jnp.float32)

    idv = ids_ref[0]
    x = x_ref[...]
    first = jnp.min(idv)
    last = jnp.max(idv)
    wiota = lax.broadcasted_iota(jnp.int32, (W, R), 0)

    def cond(ws):
        return ws <= last

    def body(ws):
        lws = jnp.minimum(ws, S - W)
        oh = (idv + jnp.zeros((W, R), jnp.int32) == lws + wiota)
        win = jax.lax.dot_general(
            oh.astype(jnp.float32), x, (((1,), (0,)), ((), ())),
            precision=jax.lax.Precision.HIGHEST,
            preferred_element_type=jnp.float32)
        o_ref[pl.ds(lws, W), :] += win
        nxt = jnp.min(jnp.where(idv >= lws + W, idv, jnp.int32(S + W)))
        return nxt

    lax.while_loop(cond, body, first)


def _tc_partial(ids_tc, rows_tc):
    return pl.pallas_call(
        _tc_body,
        grid=(B,),
        in_specs=[
            pl.BlockSpec((1, 1, R), lambda i: (i, 0, 0)),
            pl.BlockSpec((R, D), lambda i: (NSC // R + i, 0)),
        ],
        out_specs=pl.BlockSpec((S, D), lambda i: (0, 0)),
        out_shape=jax.ShapeDtypeStruct((S, D), jnp.float32),
    )(ids_tc, rows_tc)


def _combine_body(p_ref, t_ref, o_ref):
    o_ref[...] = p_ref[0] + p_ref[1] + t_ref[...]


def _combine(partials, ptc):
    blk = 256
    return pl.pallas_call(
        _combine_body,
        grid=(S // blk,),
        in_specs=[
            pl.BlockSpec((NC, blk, D), lambda i: (0, i, 0)),
            pl.BlockSpec((blk, D), lambda i: (i, 0)),
        ],
        out_specs=pl.BlockSpec((blk, D), lambda i: (i, 0)),
        out_shape=jax.ShapeDtypeStruct((S, D), jnp.float32),
    )(partials, ptc)


def kernel(node_features, batch, ptr):
    partials = _segment_sum_sc(
        node_features, batch[:NSC].reshape(NW, NCHUNKS, C))
    ptc = _tc_partial(batch[NSC:].reshape(B, 1, R), node_features)
    return _combine(partials, ptc)


# speedup vs baseline: 1.8574x; 1.0003x over previous
import functools

import jax
import jax.numpy as jnp
from jax import lax
from jax.experimental import pallas as pl
from jax.experimental.pallas import tpu as pltpu
import jax.experimental.pallas.tpu_sc as plsc

N = 320000
D = 128
S = 2048
NC = 2
NS = 16
NW = NC * NS
C = 80
NCHUNKS = 60
RPT = NCHUNKS * C
NSC = NW * RPT
NBUF = 5
PD = 3
NOUTER = NCHUNKS // NBUF
SROWS = S // NS
LANES = 16

R = 6400
B = (N - NSC) // R
W = 48


def _tile_body(nodes_hbm, batch2d_hbm, pout_hbm,
               rows_v, ids_v, zbuf_v, acc_sh, *sems):
    rsems = sems[:NBUF]
    ssems = sems[NBUF:]
    sid = lax.axis_index("s")
    cid = lax.axis_index("c")
    wid = cid * NS + sid
    row0 = wid * RPT

    def chunk_base(k):
        return pl.multiple_of(jnp.minimum(row0 + k * C, N - C), 8)

    def fetch(k, b):
        pltpu.async_copy(nodes_hbm.at[pl.ds(chunk_base(k), C)],
                         rows_v.at[b], rsems[b])

    def wait_fetch(k, b):
        pltpu.make_async_copy(
            nodes_hbm.at[pl.ds(chunk_base(k), C)],
            rows_v.at[b], rsems[b]).wait()

    def scatter(k, b):
        pltpu.async_copy(rows_v.at[b], acc_sh.at[ids_v.at[k]],
                         ssems[b], add=True)

    def wait_scatter(k, b):
        pltpu.make_async_copy(
            rows_v.at[b], acc_sh.at[ids_v.at[k]], ssems[b]).wait()

    pltpu.sync_copy(batch2d_hbm.at[wid], ids_v)
    for b in range(PD):
        fetch(jnp.int32(b), b)

    zz = jnp.zeros((LANES,), jnp.float32)

    def zero_row(i, carry):
        for j in range(D // LANES):
            zbuf_v[i, pl.ds(j * LANES, LANES)] = zz
        return carry

    lax.fori_loop(0, SROWS, zero_row, 0)
    pltpu.sync_copy(zbuf_v, acc_sh.at[pl.ds(sid * SROWS, SROWS)])
    plsc.subcore_barrier()

    def outer(k0, carry):
        for b in range(NBUF):
            k = k0 * NBUF + b
            wait_fetch(k, b)
            b2 = (b + PD) % NBUF
            if b in (0, 1):
                @pl.when(k0 > 0)
                def _():
                    wait_scatter(k - 2, b2)
            else:
                wait_scatter(k - 2, b2)
            scatter(k, b)
            fetch(k + PD, b2)
        return carry

    lax.fori_loop(0, NOUTER, outer, 0)

    for b in range(PD):
        wait_fetch(NCHUNKS + b, b)
    wait_scatter(NCHUNKS - 2, NBUF - 2)
    wait_scatter(NCHUNKS - 1, NBUF - 1)

    plsc.subcore_barrier()
    pltpu.sync_copy(acc_sh.at[pl.ds(sid * SROWS, SROWS)],
                    pout_hbm.at[cid].at[pl.ds(sid * SROWS, SROWS)])


@functools.partial(
    pl.kernel,
    out_type=jax.ShapeDtypeStruct((NC, S, D), jnp.float32),
    mesh=plsc.VectorSubcoreMesh(core_axis_name="c", subcore_axis_name="s"),
    scratch_types=[
        pltpu.VMEM((NBUF, C, D), jnp.float32),
        pltpu.VMEM((NCHUNKS, C), jnp.int32),
        pltpu.VMEM((SROWS, D), jnp.float32),
        pltpu.MemorySpace.VMEM_SHARED((S, D), jnp.float32),
    ] + [pltpu.SemaphoreType.DMA] * (2 * NBUF),
)
def _segment_sum_sc(nodes_hbm, batch2d_hbm, pout_hbm,
                    rows_v, ids_v, zbuf_v, acc_sh, *sems):
    _tile_body(nodes_hbm, batch2d_hbm, pout_hbm,
               rows_v, ids_v, zbuf_v, acc_sh, *sems)


def _tc_body(ids_ref, x_ref, o_ref):
    @pl.when(pl.program_id(0) == 0)
    def _():
        o_ref[...] = jnp.zeros((S, D), jnp.float32)

    idv = ids_ref[0]
    x = x_ref[...]
    first = jnp.min(idv)
    last = jnp.max(idv)
    wiota = lax.broadcasted_iota(jnp.int32, (W, R), 0)

    def cond(ws):
        return ws <= last

    xhi = x.astype(jnp.bfloat16).astype(jnp.float32)
    xlo = x - xhi

    def body(ws):
        lws = jnp.minimum(ws, S - W)
        oh = (idv + jnp.zeros((W, R), jnp.int32) == lws + wiota)
        ohf = oh.astype(jnp.float32)
        dims = (((1,), (0,)), ((), ()))
        win = (jax.lax.dot_general(ohf, xhi, dims,
                                   preferred_element_type=jnp.float32)
               + jax.lax.dot_general(ohf, xlo, dims,
                                     preferred_element_type=jnp.float32))
        o_ref[pl.ds(lws, W), :] += win
        nxt = jnp.min(jnp.where(idv >= lws + W, idv, jnp.int32(S + W)))
        return nxt

    lax.while_loop(cond, body, first)


def _tc_partial(ids_tc, rows_tc):
    return pl.pallas_call(
        _tc_body,
        grid=(B,),
        in_specs=[
            pl.BlockSpec((1, 1, R), lambda i: (i, 0, 0)),
            pl.BlockSpec((R, D), lambda i: (NSC // R + i, 0)),
        ],
        out_specs=pl.BlockSpec((S, D), lambda i: (0, 0)),
        out_shape=jax.ShapeDtypeStruct((S, D), jnp.float32),
    )(ids_tc, rows_tc)


def _combine_body(p_ref, t_ref, o_ref):
    o_ref[...] = p_ref[0] + p_ref[1] + t_ref[...]


def _combine(partials, ptc):
    blk = 256
    return pl.pallas_call(
        _combine_body,
        grid=(S // blk,),
        in_specs=[
            pl.BlockSpec((NC, blk, D), lambda i: (0, i, 0)),
            pl.BlockSpec((blk, D), lambda i: (i, 0)),
        ],
        out_specs=pl.BlockSpec((blk, D), lambda i: (i, 0)),
        out_shape=jax.ShapeDtypeStruct((S, D), jnp.float32),
    )(partials, ptc)


def kernel(node_features, batch, ptr):
    partials = _segment_sum_sc(
        node_features, batch[:NSC].reshape(NW, NCHUNKS, C))
    ptc = _tc_partial(batch[NSC:].reshape(B, 1, R), node_features)
    return _combine(partials, ptc)


# speedup vs baseline: 1.9034x; 1.0248x over previous
import functools

import jax
import jax.numpy as jnp
from jax import lax
from jax.experimental import pallas as pl
from jax.experimental.pallas import tpu as pltpu
import jax.experimental.pallas.tpu_sc as plsc

N = 320000
D = 128
S = 2048
NC = 2
NS = 16
NW = NC * NS
C = 80
NCHUNKS = 50
RPT = NCHUNKS * C
NSC = NW * RPT
NBUF = 5
PD = 3
NOUTER = NCHUNKS // NBUF
SROWS = S // NS
LANES = 16

R = 6400
B = (N - NSC) // R
W = 48


def _tile_body(nodes_hbm, batch2d_hbm, pout_hbm,
               rows_v, ids_v, zbuf_v, acc_sh, *sems):
    rsems = sems[:NBUF]
    ssems = sems[NBUF:]
    sid = lax.axis_index("s")
    cid = lax.axis_index("c")
    wid = cid * NS + sid
    row0 = wid * RPT

    def chunk_base(k):
        return pl.multiple_of(jnp.minimum(row0 + k * C, N - C), 8)

    def fetch(k, b):
        pltpu.async_copy(nodes_hbm.at[pl.ds(chunk_base(k), C)],
                         rows_v.at[b], rsems[b])

    def wait_fetch(k, b):
        pltpu.make_async_copy(
            nodes_hbm.at[pl.ds(chunk_base(k), C)],
            rows_v.at[b], rsems[b]).wait()

    def scatter(k, b):
        pltpu.async_copy(rows_v.at[b], acc_sh.at[ids_v.at[k]],
                         ssems[b], add=True)

    def wait_scatter(k, b):
        pltpu.make_async_copy(
            rows_v.at[b], acc_sh.at[ids_v.at[k]], ssems[b]).wait()

    pltpu.sync_copy(batch2d_hbm.at[wid], ids_v)
    for b in range(PD):
        fetch(jnp.int32(b), b)

    zz = jnp.zeros((LANES,), jnp.float32)

    def zero_row(i, carry):
        for j in range(D // LANES):
            zbuf_v[i, pl.ds(j * LANES, LANES)] = zz
        return carry

    lax.fori_loop(0, SROWS, zero_row, 0)
    pltpu.sync_copy(zbuf_v, acc_sh.at[pl.ds(sid * SROWS, SROWS)])
    plsc.subcore_barrier()

    def outer(k0, carry):
        for b in range(NBUF):
            k = k0 * NBUF + b
            wait_fetch(k, b)
            b2 = (b + PD) % NBUF
            if b in (0, 1):
                @pl.when(k0 > 0)
                def _():
                    wait_scatter(k - 2, b2)
            else:
                wait_scatter(k - 2, b2)
            scatter(k, b)
            fetch(k + PD, b2)
        return carry

    lax.fori_loop(0, NOUTER, outer, 0)

    for b in range(PD):
        wait_fetch(NCHUNKS + b, b)
    wait_scatter(NCHUNKS - 2, NBUF - 2)
    wait_scatter(NCHUNKS - 1, NBUF - 1)

    plsc.subcore_barrier()
    pltpu.sync_copy(acc_sh.at[pl.ds(sid * SROWS, SROWS)],
                    pout_hbm.at[cid].at[pl.ds(sid * SROWS, SROWS)])


@functools.partial(
    pl.kernel,
    out_type=jax.ShapeDtypeStruct((NC, S, D), jnp.float32),
    mesh=plsc.VectorSubcoreMesh(core_axis_name="c", subcore_axis_name="s"),
    scratch_types=[
        pltpu.VMEM((NBUF, C, D), jnp.float32),
        pltpu.VMEM((NCHUNKS, C), jnp.int32),
        pltpu.VMEM((SROWS, D), jnp.float32),
        pltpu.MemorySpace.VMEM_SHARED((S, D), jnp.float32),
    ] + [pltpu.SemaphoreType.DMA] * (2 * NBUF),
)
def _segment_sum_sc(nodes_hbm, batch2d_hbm, pout_hbm,
                    rows_v, ids_v, zbuf_v, acc_sh, *sems):
    _tile_body(nodes_hbm, batch2d_hbm, pout_hbm,
               rows_v, ids_v, zbuf_v, acc_sh, *sems)


def _tc_body(ids_ref, x_ref, o_ref):
    @pl.when(pl.program_id(0) == 0)
    def _():
        o_ref[...] = jnp.zeros((S, D), jnp.float32)

    idv = ids_ref[0]
    x = x_ref[...]
    first = jnp.min(idv)
    last = jnp.max(idv)
    wiota = lax.broadcasted_iota(jnp.int32, (W, R), 0)

    def cond(ws):
        return ws <= last

    def body(ws):
        lws = jnp.minimum(ws, S - W)
        oh = (idv + jnp.zeros((W, R), jnp.int32) == lws + wiota)
        win = jax.lax.dot_general(
            oh.astype(jnp.float32), x, (((1,), (0,)), ((), ())),
            preferred_element_type=jnp.float32)
        o_ref[pl.ds(lws, W), :] += win
        nxt = jnp.min(jnp.where(idv >= lws + W, idv, jnp.int32(S + W)))
        return nxt

    lax.while_loop(cond, body, first)


def _tc_partial(ids_tc, rows_tc):
    return pl.pallas_call(
        _tc_body,
        grid=(B,),
        in_specs=[
            pl.BlockSpec((1, 1, R), lambda i: (i, 0, 0)),
            pl.BlockSpec((R, D), lambda i: (NSC // R + i, 0)),
        ],
        out_specs=pl.BlockSpec((S, D), lambda i: (0, 0)),
        out_shape=jax.ShapeDtypeStruct((S, D), jnp.float32),
    )(ids_tc, rows_tc)


def _combine_body(p_ref, t_ref, o_ref):
    o_ref[...] = p_ref[0] + p_ref[1] + t_ref[...]


def _combine(partials, ptc):
    blk = 256
    return pl.pallas_call(
        _combine_body,
        grid=(S // blk,),
        in_specs=[
            pl.BlockSpec((NC, blk, D), lambda i: (0, i, 0)),
            pl.BlockSpec((blk, D), lambda i: (i, 0)),
        ],
        out_specs=pl.BlockSpec((blk, D), lambda i: (i, 0)),
        out_shape=jax.ShapeDtypeStruct((S, D), jnp.float32),
    )(partials, ptc)


def kernel(node_features, batch, ptr):
    partials = _segment_sum_sc(
        node_features, batch[:NSC].reshape(NW, NCHUNKS, C))
    ptc = _tc_partial(batch[NSC:].reshape(B, 1, R), node_features)
    return _combine(partials, ptc)
